# baseline (speedup 1.0000x reference)
import functools

import jax
import jax.numpy as jnp
from jax import lax
from jax.experimental import pallas as pl
from jax.experimental.pallas import tpu as pltpu
from jax.experimental.pallas import tpu_sc as plsc

def _gelu(x):
    return 0.5 * x * (1.0 + lax.erf(x * 0.7071067811865476))


_N = 10000
_D = 128
_NPAD = 10240
_NSUB = 16
_CHUNK = 128


def _spmm_call(table, idx_packed, n_chunks):
    rpt = _NPAD // _NSUB
    npairs = n_chunks // 2

    def body(table_hbm, idx_hbm, zero_hbm, out_hbm,
             idx0, idx1, rows0, rows1, acc_sh, gsem0, gsem1):
        c = lax.axis_index("c")
        s = lax.axis_index("s")
        pltpu.sync_copy(zero_hbm, acc_sh.at[pl.ds(s * rpt, rpt)])
        plsc.subcore_barrier()

        row0 = s * n_chunks
        pltpu.sync_copy(idx_hbm.at[c, row0], idx0)
        pltpu.async_copy(table_hbm.at[idx0.at[0]], rows0, gsem0)

        def step(j, carry):
            pltpu.sync_copy(idx_hbm.at[c, row0 + 2 * j + 1], idx1)
            pltpu.make_async_copy(table_hbm.at[idx0.at[0]], rows0, gsem0).wait()
            pltpu.async_copy(table_hbm.at[idx1.at[0]], rows1, gsem1)
            pltpu.sync_copy(rows0, acc_sh.at[idx0.at[1]], add=True)

            @pl.when(j < npairs - 1)
            def _():
                pltpu.sync_copy(idx_hbm.at[c, row0 + 2 * j + 2], idx0)
                pltpu.async_copy(table_hbm.at[idx0.at[0]], rows0, gsem0)

            pltpu.make_async_copy(table_hbm.at[idx1.at[0]], rows1, gsem1).wait()
            pltpu.sync_copy(rows1, acc_sh.at[idx1.at[1]], add=True)
            return carry

        lax.fori_loop(0, npairs, step, 0)
        plsc.subcore_barrier()
        pltpu.sync_copy(acc_sh.at[pl.ds(s * rpt, rpt)],
                        out_hbm.at[c, pl.ds(s * rpt, rpt)])

    mesh = plsc.VectorSubcoreMesh(core_axis_name="c", subcore_axis_name="s")
    f = pl.kernel(
        body,
        out_type=jax.ShapeDtypeStruct((2, _NPAD, _D), jnp.float32),
        mesh=mesh,
        scratch_types=[
            pltpu.VMEM((2, _CHUNK), jnp.int32),
            pltpu.VMEM((2, _CHUNK), jnp.int32),
            pltpu.VMEM((_CHUNK, _D), jnp.float32),
            pltpu.VMEM((_CHUNK, _D), jnp.float32),
            pltpu.VMEM_SHARED((_NPAD, _D), jnp.float32),
            pltpu.SemaphoreType.DMA,
            pltpu.SemaphoreType.DMA,
        ],
    )
    zero = jnp.zeros((rpt, _D), jnp.float32)
    return f(table, idx_packed, zero)


def _tc_mm(x, w, b, act):
    m = x.shape[0]
    bm = 2048
    assert m % bm == 0

    def body(x_ref, w_ref, b_ref, o_ref):
        xv = x_ref[...]
        if act:
            xv = _gelu(xv + b_ref[...])
        o_ref[...] = jnp.dot(xv, w_ref[...], preferred_element_type=jnp.float32)

    return pl.pallas_call(
        body,
        grid=(m // bm,),
        in_specs=[
            pl.BlockSpec((bm, _D), lambda i: (i, 0)),
            pl.BlockSpec((_D, _D), lambda i: (0, 0)),
            pl.BlockSpec((1, _D), lambda i: (0, 0)),
        ],
        out_specs=pl.BlockSpec((bm, _D), lambda i: (i, 0)),
        out_shape=jax.ShapeDtypeStruct((m, _D), jnp.float32),
    )(x, w, b.reshape(1, _D))


def _tc_std(s2, b):
    def body(x_ref, b_ref, o_ref):
        x = x_ref[0] + b_ref[...]
        x = _gelu(x)
        mu = jnp.mean(x, axis=0, keepdims=True)
        xc = x - mu
        var = jnp.sum(xc * xc, axis=0, keepdims=True) / (_N - 1)
        o_ref[0] = xc * lax.rsqrt(var)

    return pl.pallas_call(
        body,
        grid=(2,),
        in_specs=[
            pl.BlockSpec((1, _N, _D), lambda g: (g, 0, 0)),
            pl.BlockSpec((1, _D), lambda g: (0, 0)),
        ],
        out_specs=pl.BlockSpec((1, _N, _D), lambda g: (g, 0, 0)),
        out_shape=jax.ShapeDtypeStruct((2, _N, _D), jnp.float32),
    )(s2, b.reshape(1, _D))


def kernel(X_a, edge_index_a, X_b, edge_index_b, W0, b0, W1, b1):
    e = edge_index_a.shape[1]
    n_chunks = 4 * (-(-e // (_NSUB * _CHUNK * 4)))
    ep = _NSUB * n_chunks * _CHUNK

    def prep(ei, coff):
        pad = ep - e
        src = jnp.concatenate([ei[0], jnp.zeros((pad,), jnp.int32)]) + coff
        dst = jnp.concatenate([ei[1], jnp.full((pad,), _N, jnp.int32)])
        return jnp.stack([src.reshape(_NSUB * n_chunks, _CHUNK),
                          dst.reshape(_NSUB * n_chunks, _CHUNK)], axis=1)

    idx = jnp.stack([prep(edge_index_a, 0), prep(edge_index_b, _NPAD)])

    xp = jnp.zeros((2, _NPAD, _D), jnp.float32)
    xp = xp.at[0, :_N].set(X_a).at[1, :_N].set(X_b)

    h = _tc_mm(xp.reshape(2 * _NPAD, _D), W0, b0, act=False)
    s1 = _spmm_call(h, idx, n_chunks)
    h2 = _tc_mm(s1.reshape(2 * _NPAD, _D), W1, b0, act=True)
    s2 = _spmm_call(h2, idx, n_chunks)
    out = _tc_std(s2[:, :_N], b1)
    return (out[0], out[1])


# speedup vs baseline: 2.3379x; 2.3379x over previous
import functools

import jax
import jax.numpy as jnp
from jax import lax
from jax.experimental import pallas as pl
from jax.experimental.pallas import tpu as pltpu
from jax.experimental.pallas import tpu_sc as plsc

def _gelu(x):
    return 0.5 * x * (1.0 + lax.erf(x * 0.7071067811865476))


_N = 10000
_D = 128
_NPAD = 10240
_NSUB = 16
_CHUNK = 128


def _spmm_call(table, idx_packed, n_chunks):
    rpt = _NPAD // _NSUB
    npairs = n_chunks // 2

    def body(table_hbm, idx_hbm, zero_hbm, out_hbm,
             idx0, idx1, rows0, rows1, acc_sh, gsem0, gsem1):
        c = lax.axis_index("c")
        s = lax.axis_index("s")
        pltpu.sync_copy(zero_hbm, acc_sh.at[pl.ds(s * rpt, rpt)])
        plsc.subcore_barrier()

        row0 = s * n_chunks
        pltpu.sync_copy(idx_hbm.at[c, row0], idx0)
        pltpu.async_copy(table_hbm.at[idx0.at[0]], rows0, gsem0)

        def step(j, carry):
            pltpu.sync_copy(idx_hbm.at[c, row0 + 2 * j + 1], idx1)
            pltpu.make_async_copy(table_hbm.at[idx0.at[0]], rows0, gsem0).wait()
            pltpu.async_copy(table_hbm.at[idx1.at[0]], rows1, gsem1)
            pltpu.sync_copy(rows0, acc_sh.at[idx0.at[1]], add=True)

            @pl.when(j < npairs - 1)
            def _():
                pltpu.sync_copy(idx_hbm.at[c, row0 + 2 * j + 2], idx0)
                pltpu.async_copy(table_hbm.at[idx0.at[0]], rows0, gsem0)

            pltpu.make_async_copy(table_hbm.at[idx1.at[0]], rows1, gsem1).wait()
            pltpu.sync_copy(rows1, acc_sh.at[idx1.at[1]], add=True)
            return carry

        lax.fori_loop(0, npairs, step, 0)
        plsc.subcore_barrier()
        pltpu.sync_copy(acc_sh.at[pl.ds(s * rpt, rpt)],
                        out_hbm.at[c, pl.ds(s * rpt, rpt)])

    mesh = plsc.VectorSubcoreMesh(core_axis_name="c", subcore_axis_name="s")
    f = pl.kernel(
        body,
        out_type=jax.ShapeDtypeStruct((2, _NPAD, _D), jnp.float32),
        mesh=mesh,
        scratch_types=[
            pltpu.VMEM((2, _CHUNK), jnp.int32),
            pltpu.VMEM((2, _CHUNK), jnp.int32),
            pltpu.VMEM((_CHUNK, _D), jnp.float32),
            pltpu.VMEM((_CHUNK, _D), jnp.float32),
            pltpu.VMEM_SHARED((_NPAD, _D), jnp.float32),
            pltpu.SemaphoreType.DMA,
            pltpu.SemaphoreType.DMA,
        ],
    )
    zero = jnp.zeros((rpt, _D), jnp.float32)
    return f(table, idx_packed, zero)


def _tc_mm(x, w, b, act):
    m = x.shape[0]
    bm = 2048
    assert m % bm == 0

    def body(x_ref, w_ref, b_ref, o_ref):
        xv = x_ref[...]
        if act:
            xv = _gelu(xv + b_ref[...])
        o_ref[...] = jnp.dot(xv, w_ref[...], preferred_element_type=jnp.float32)

    return pl.pallas_call(
        body,
        grid=(m // bm,),
        in_specs=[
            pl.BlockSpec((bm, _D), lambda i: (i, 0)),
            pl.BlockSpec((_D, _D), lambda i: (0, 0)),
            pl.BlockSpec((1, _D), lambda i: (0, 0)),
        ],
        out_specs=pl.BlockSpec((bm, _D), lambda i: (i, 0)),
        out_shape=jax.ShapeDtypeStruct((m, _D), jnp.float32),
    )(x, w, b.reshape(1, _D))


def _tc_std(s2, b):
    def body(x_ref, b_ref, o_ref):
        x = x_ref[0] + b_ref[...]
        x = _gelu(x)
        mu = jnp.mean(x, axis=0, keepdims=True)
        xc = x - mu
        var = jnp.sum(xc * xc, axis=0, keepdims=True) / (_N - 1)
        o_ref[0] = xc * lax.rsqrt(var)

    return pl.pallas_call(
        body,
        grid=(2,),
        in_specs=[
            pl.BlockSpec((1, _N, _D), lambda g: (g, 0, 0)),
            pl.BlockSpec((1, _D), lambda g: (0, 0)),
        ],
        out_specs=pl.BlockSpec((1, _N, _D), lambda g: (g, 0, 0)),
        out_shape=jax.ShapeDtypeStruct((2, _N, _D), jnp.float32),
    )(s2, b.reshape(1, _D))


def kernel(X_a, edge_index_a, X_b, edge_index_b, W0, b0, W1, b1):
    e = edge_index_a.shape[1]
    n_chunks = 2 * (-(-e // (_NSUB * _CHUNK * 2)))
    ep = _NSUB * n_chunks * _CHUNK
    pad = ep - e
    pad_src = (jnp.arange(pad, dtype=jnp.int32) * 97) % _N
    pad_dst = _N + (jnp.arange(pad, dtype=jnp.int32) % (_NPAD - _N))

    def prep(ei, coff):
        src = jnp.concatenate([ei[0], pad_src]) + coff
        dst = jnp.concatenate([ei[1], pad_dst])
        return jnp.stack([src.reshape(_NSUB * n_chunks, _CHUNK),
                          dst.reshape(_NSUB * n_chunks, _CHUNK)], axis=1)

    idx = jnp.stack([prep(edge_index_a, 0), prep(edge_index_b, _NPAD)])

    xp = jnp.zeros((2, _NPAD, _D), jnp.float32)
    xp = xp.at[0, :_N].set(X_a).at[1, :_N].set(X_b)

    h = _tc_mm(xp.reshape(2 * _NPAD, _D), W0, b0, act=False)
    s1 = _spmm_call(h, idx, n_chunks)
    h2 = _tc_mm(s1.reshape(2 * _NPAD, _D), W1, b0, act=True)
    s2 = _spmm_call(h2, idx, n_chunks)
    out = _tc_std(s2[:, :_N], b1)
    return (out[0], out[1])


# speedup vs baseline: 2.3805x; 1.0182x over previous
import functools

import jax
import jax.numpy as jnp
from jax import lax
from jax.experimental import pallas as pl
from jax.experimental.pallas import tpu as pltpu
from jax.experimental.pallas import tpu_sc as plsc

def _gelu(x):
    return 0.5 * x * (1.0 + lax.erf(x * 0.7071067811865476))


_N = 10000
_D = 128
_NPAD = 10112
_NSUB = 16
_CHUNK = 128


def _spmm_call(table, idx_packed, n_chunks):
    rpt = _NPAD // _NSUB
    nt = n_chunks // 3

    def body(table_hbm, idx_hbm, zero_hbm, out_hbm,
             i0, i1, i2, r0, r1, r2, acc_sh, is0, is1, is2, g0, g1, g2):
        c = lax.axis_index("c")
        s = lax.axis_index("s")
        idx = (i0, i1, i2)
        rows = (r0, r1, r2)
        isem = (is0, is1, is2)
        gsem = (g0, g1, g2)
        pltpu.sync_copy(zero_hbm, acc_sh.at[pl.ds(s * rpt, rpt)])
        plsc.subcore_barrier()

        row0 = s * n_chunks

        def idx_load(k, u):
            pltpu.async_copy(idx_hbm.at[c, row0 + k], idx[u], isem[u])

        def idx_wait(k, u):
            pltpu.make_async_copy(idx_hbm.at[c, row0 + k], idx[u],
                                  isem[u]).wait()

        def gth(u):
            pltpu.async_copy(table_hbm.at[idx[u].at[0]], rows[u], gsem[u])

        def gth_wait(u):
            pltpu.make_async_copy(table_hbm.at[idx[u].at[0]], rows[u],
                                  gsem[u]).wait()

        for u in range(3):
            idx_load(u, u)
        idx_wait(0, 0)
        gth(0)
        idx_wait(1, 1)
        gth(1)

        def step(t, carry):
            for u in range(3):
                k = 3 * t + u
                u2 = (u + 2) % 3
                gth_wait(u)

                @pl.when(k + 2 < n_chunks)
                def _():
                    idx_wait(k + 2, u2)
                    gth(u2)

                pltpu.sync_copy(rows[u], acc_sh.at[idx[u].at[1]], add=True)

                @pl.when(k + 3 < n_chunks)
                def _():
                    idx_load(k + 3, u)
            return carry

        lax.fori_loop(0, nt, step, 0)
        plsc.subcore_barrier()
        pltpu.sync_copy(acc_sh.at[pl.ds(s * rpt, rpt)],
                        out_hbm.at[c, pl.ds(s * rpt, rpt)])

    mesh = plsc.VectorSubcoreMesh(core_axis_name="c", subcore_axis_name="s")
    f = pl.kernel(
        body,
        out_type=jax.ShapeDtypeStruct((2, _NPAD, _D), jnp.float32),
        mesh=mesh,
        scratch_types=[
            pltpu.VMEM((2, _CHUNK), jnp.int32),
            pltpu.VMEM((2, _CHUNK), jnp.int32),
            pltpu.VMEM((2, _CHUNK), jnp.int32),
            pltpu.VMEM((_CHUNK, _D), jnp.float32),
            pltpu.VMEM((_CHUNK, _D), jnp.float32),
            pltpu.VMEM((_CHUNK, _D), jnp.float32),
            pltpu.VMEM_SHARED((_NPAD, _D), jnp.float32),
        ] + [pltpu.SemaphoreType.DMA] * 6,
    )
    zero = jnp.zeros((rpt, _D), jnp.float32)
    return f(table, idx_packed, zero)


def _tc_mm(x, w, b, act):
    m = x.shape[0]
    bm = m // 8
    assert m % 8 == 0 and bm % 8 == 0

    def body(x_ref, w_ref, b_ref, o_ref):
        xv = x_ref[...]
        if act:
            xv = _gelu(xv + b_ref[...])
        o_ref[...] = jnp.dot(xv, w_ref[...], preferred_element_type=jnp.float32)

    return pl.pallas_call(
        body,
        grid=(m // bm,),
        in_specs=[
            pl.BlockSpec((bm, _D), lambda i: (i, 0)),
            pl.BlockSpec((_D, _D), lambda i: (0, 0)),
            pl.BlockSpec((1, _D), lambda i: (0, 0)),
        ],
        out_specs=pl.BlockSpec((bm, _D), lambda i: (i, 0)),
        out_shape=jax.ShapeDtypeStruct((m, _D), jnp.float32),
    )(x, w, b.reshape(1, _D))


def _tc_std(s2, b):
    def body(x_ref, b_ref, o_ref):
        x = x_ref[0] + b_ref[...]
        x = _gelu(x)
        mu = jnp.mean(x, axis=0, keepdims=True)
        xc = x - mu
        var = jnp.sum(xc * xc, axis=0, keepdims=True) / (_N - 1)
        o_ref[0] = xc * lax.rsqrt(var)

    return pl.pallas_call(
        body,
        grid=(2,),
        in_specs=[
            pl.BlockSpec((1, _N, _D), lambda g: (g, 0, 0)),
            pl.BlockSpec((1, _D), lambda g: (0, 0)),
        ],
        out_specs=pl.BlockSpec((1, _N, _D), lambda g: (g, 0, 0)),
        out_shape=jax.ShapeDtypeStruct((2, _N, _D), jnp.float32),
    )(s2, b.reshape(1, _D))


def kernel(X_a, edge_index_a, X_b, edge_index_b, W0, b0, W1, b1):
    e = edge_index_a.shape[1]
    n_chunks = 3 * (-(-e // (_NSUB * _CHUNK * 3)))
    ep = _NSUB * n_chunks * _CHUNK
    pad = ep - e
    pad_src = ((jnp.arange(pad, dtype=jnp.int32) * 97) % _N
               ).reshape(_NSUB, pad // _NSUB)
    pad_dst = (_N + (jnp.arange(pad, dtype=jnp.int32) % (_NPAD - _N))
               ).reshape(_NSUB, pad // _NSUB)

    def prep(ei, coff):
        src = jnp.concatenate([ei[0].reshape(_NSUB, e // _NSUB),
                               pad_src], axis=1) + coff
        dst = jnp.concatenate([ei[1].reshape(_NSUB, e // _NSUB),
                               pad_dst], axis=1)
        return jnp.stack([src.reshape(_NSUB * n_chunks, _CHUNK),
                          dst.reshape(_NSUB * n_chunks, _CHUNK)], axis=1)

    idx = jnp.stack([prep(edge_index_a, 0), prep(edge_index_b, _NPAD)])

    xp = jnp.zeros((2, _NPAD, _D), jnp.float32)
    xp = xp.at[0, :_N].set(X_a).at[1, :_N].set(X_b)

    h = _tc_mm(xp.reshape(2 * _NPAD, _D), W0, b0, act=False)
    s1 = _spmm_call(h, idx, n_chunks)
    h2 = _tc_mm(s1.reshape(2 * _NPAD, _D), W1, b0, act=True)
    s2 = _spmm_call(h2, idx, n_chunks)
    out = _tc_std(s2[:, :_N], b1)
    return (out[0], out[1])
